# baseline (speedup 1.0000x reference)
import functools

import jax
import jax.numpy as jnp
import numpy as np
from jax import lax
from jax.experimental import pallas as pl
from jax.experimental.pallas import tpu as pltpu
from jax.experimental.pallas import tpu_sc as plsc

_PRIMES = (499801, 499819, 499853, 499879, 499883, 499897, 499903, 499927,
           499943, 499957, 499969, 499973, 499979, 500009, 500029, 500041)
_NUM_TABLES = 16
_EMBED_DIM = 16
_HIDDEN = 1024
_ORDERS = tuple([2] * 8 + [3] * 8)
_OFFSETS = tuple(np.concatenate([[0], np.cumsum(_PRIMES)[:-1]]).astype(np.int64))

_NC, _NS = 2, 16
_NW = _NC * _NS


def _mod_prime(x, p):
    q = (x.astype(jnp.float32) * np.float32(1.0 / p)).astype(jnp.int32)
    r = x - q * np.int32(p)
    r = jnp.where(r < 0, r + np.int32(p), r)
    r = jnp.where(r >= np.int32(p), r - np.int32(p), r)
    return r


def _hash_kernel(tok0, tok1, tok2, mults, bias, out_ref):
    toks = (tok0, tok1, tok2)
    mask16 = np.int32(0xFFFF)
    _S16 = np.int32(16)
    for t in range(_NUM_TABLES):
        order = _ORDERS[t]
        l0 = jnp.zeros_like(tok0[...])
        l1 = jnp.zeros_like(l0)
        l2 = jnp.zeros_like(l0)
        for p in range(order):
            m = mults[t, p]
            m_lo = m & mask16
            m_hi = lax.shift_right_logical(m, _S16)
            tv = toks[p][...]
            p_lo = m_lo * tv
            p_hi = m_hi * tv
            q0 = p_lo & mask16
            mid = lax.shift_right_logical(p_lo, _S16) + (p_hi & mask16)
            q1 = mid & mask16
            q2 = lax.shift_right_logical(p_hi, _S16) + lax.shift_right_logical(mid, _S16)
            l0 = l0 ^ q0
            l1 = l1 ^ q1
            l2 = l2 ^ q2
        b = bias[t]
        l0 = l0 ^ (b & mask16)
        l1 = l1 ^ lax.shift_right_logical(b, _S16)
        p = _PRIMES[t]
        r1 = _mod_prime(l2 * np.int32(65536) + l1, p)
        r2 = _mod_prime(r1 * np.int32(4096), p)
        idx = _mod_prime(r2 * np.int32(16) + l0, p) + np.int32(_OFFSETS[t])
        out_ref[t] = idx


def _compute_indices(tok0, tok1, tok2, mults, bias):
    B, S = tok0.shape
    return pl.pallas_call(
        _hash_kernel,
        out_shape=jax.ShapeDtypeStruct((_NUM_TABLES, B, S), jnp.int32),
        in_specs=[
            pl.BlockSpec(memory_space=pltpu.VMEM),
            pl.BlockSpec(memory_space=pltpu.VMEM),
            pl.BlockSpec(memory_space=pltpu.VMEM),
            pl.BlockSpec(memory_space=pltpu.SMEM),
            pl.BlockSpec(memory_space=pltpu.SMEM),
        ],
        out_specs=pl.BlockSpec(memory_space=pltpu.VMEM),
        name="ngram_hash",
    )(tok0, tok1, tok2, mults, bias)


def _gather_body(rows_per_worker, chunk, table_hbm, idx_hbm, out_hbm,
                 idx_v, rows_v, sem):
    wid = lax.axis_index("s") * np.int32(_NC) + lax.axis_index("c")
    base = wid * np.int32(rows_per_worker)
    pltpu.sync_copy(idx_hbm.at[pl.ds(base, rows_per_worker)], idx_v)
    nch = rows_per_worker // chunk

    for j in range(nch):
        off = j * chunk
        pltpu.make_async_copy(
            table_hbm.at[idx_v.at[pl.ds(off, chunk)]],
            rows_v.at[pl.ds(off, chunk)],
            sem,
        ).start()
    pltpu.make_async_copy(
        table_hbm.at[pl.ds(0, rows_per_worker)], rows_v, sem).wait()
    pltpu.sync_copy(rows_v, out_hbm.at[pl.ds(base, rows_per_worker)])


def _gather_rows(table, idx_flat):
    n = idx_flat.shape[0]
    rpw = n // _NW
    chunk = 128
    mesh = plsc.VectorSubcoreMesh(core_axis_name="c", subcore_axis_name="s")
    k = pl.kernel(
        functools.partial(_gather_body, rpw, chunk),
        out_type=jax.ShapeDtypeStruct((n, _EMBED_DIM), jnp.float32),
        mesh=mesh,
        name="sc_gather",
        compiler_params=pltpu.CompilerParams(use_tc_tiling_on_sc=False),
        scratch_types=[
            pltpu.VMEM((rpw,), jnp.int32),
            pltpu.VMEM((rpw, _EMBED_DIM), jnp.float32),
            pltpu.SemaphoreType.DMA,
        ],
    )
    return k(table, idx_flat)


def _matmul_kernel(emb_ref, w_ref, out_ref):
    out_ref[...] = lax.dot_general(
        emb_ref[...], w_ref[...],
        (((1,), (1,)), ((), ())),
        preferred_element_type=jnp.float32)


def _project(emb, w_out):
    n = emb.shape[0]
    blk = 1024
    return pl.pallas_call(
        _matmul_kernel,
        grid=(n // blk,),
        in_specs=[
            pl.BlockSpec((blk, _NUM_TABLES * _EMBED_DIM),
                         lambda i: (i, np.int32(0))),
            pl.BlockSpec((_HIDDEN, _NUM_TABLES * _EMBED_DIM),
                         lambda i: (np.int32(0), np.int32(0))),
        ],
        out_specs=pl.BlockSpec((blk, _HIDDEN), lambda i: (i, np.int32(0))),
        out_shape=jax.ShapeDtypeStruct((n, _HIDDEN), jnp.float32),
        name="out_proj",
    )(emb, w_out)


def kernel(token_ids, hash_mults, hash_bias, table, w_out):
    B, S = token_ids.shape
    tok0 = token_ids.astype(jnp.int32)
    tok1 = jnp.pad(tok0[:, :S - 1], ((0, 0), (1, 0)))
    tok2 = jnp.pad(tok0[:, :S - 2], ((0, 0), (2, 0)))
    mults = hash_mults.astype(jnp.int32)
    bias = hash_bias.astype(jnp.int32)

    idx_tbs = _compute_indices(tok0, tok1, tok2, mults, bias)
    idx_flat = jnp.transpose(idx_tbs, (1, 2, 0)).reshape(-1)
    emb = _gather_rows(table, idx_flat)
    emb = emb.reshape(B * S, _NUM_TABLES * _EMBED_DIM)
    out = _project(emb, w_out)
    return out.reshape(B, S, _HIDDEN)
